# baseline (speedup 1.0000x reference)
import functools

import jax
import jax.numpy as jnp
from jax import lax
from jax.experimental import pallas as pl
from jax.experimental.pallas import tpu as pltpu
from jax.experimental.pallas import tpu_sc as plsc

N = 10000
E = 320000
H = 128
NC = 2
NS = 16
NW = NC * NS
RL = 128
CH = 64
RW = 80
EROWS = NW * RW // 2 * 2
EROWS = 2560
EPAD = EROWS * RL - E
N_SH = N + 8
RPT = 624
RPT_LAST = N - 15 * RPT
ZR = 64

_K1 = -2.0 * 0.7978845608028654
_K2 = _K1 * 0.044715

_mesh = plsc.VectorSubcoreMesh(
    core_axis_name="c", subcore_axis_name="s", num_cores=NC, num_subcores=NS)


def _gelu16(t):
  d = jnp.exp(_K1 * t + _K2 * (t * t * t))
  return t / (1.0 + d)


def _drain(sh, out, c, s):
  @pl.when(s < NS - 1)
  def _():
    pltpu.sync_copy(sh.at[pl.ds(s * RPT, RPT)],
                    out.at[c, pl.ds(s * RPT, RPT)])

  @pl.when(s == NS - 1)
  def _():
    pltpu.sync_copy(sh.at[pl.ds(15 * RPT, RPT_LAST)],
                    out.at[c, pl.ds(15 * RPT, RPT_LAST)])


def _fill(buf, rows, value):
  @pl.loop(0, rows)
  def _(i):
    for k in range(H // 16):
      buf[i, pl.ds(k * 16, 16)] = jnp.full((16,), value, jnp.float32)


def _split_row(row, a64, b64):
  for k in range(4):
    a64[pl.ds(k * 16, 16)] = row[pl.ds(k * 16, 16)]
  for k in range(4):
    b64[pl.ds(k * 16, 16)] = row[pl.ds(64 + k * 16, 16)]


@functools.partial(
    pl.kernel,
    out_type=jax.ShapeDtypeStruct((NC, N, H), jnp.float32),
    mesh=_mesh,
    scratch_types=[
        [pltpu.VMEM((RL,), jnp.int32)] * 4,
        pltpu.VMEM((RL, H), jnp.float32),
        pltpu.VMEM_SHARED((N_SH, H), jnp.float32),
        [pltpu.SemaphoreType.DMA] * 4,
        [pltpu.SemaphoreType.DMA] * 2,
    ],
)
def _sc_degree(dst2d, cnt_out, dvr, ones_z, cnt_sh, sem_s, sem_ir):
  c = lax.axis_index("c")
  s = lax.axis_index("s")
  wid = s * NC + c

  _fill(ones_z, ZR, 0.0)
  for r5 in range(10):
    pltpu.sync_copy(ones_z.at[pl.ds(0, ZR)],
                    cnt_sh.at[pl.ds(s * RPT + r5 * ZR, ZR)])
  _fill(ones_z, RL, 1.0)
  plsc.subcore_barrier()

  pltpu.sync_copy(dst2d.at[wid], dvr[0])
  pltpu.async_copy(dst2d.at[wid + NW], dvr[1], sem_ir[1])

  @pl.loop(0, RW, step=4)
  def _(tt):
    for b in range(4):
      t = tt + b

      @pl.when(t >= 1)
      def _():
        pltpu.make_async_copy(dst2d.at[wid + t * NW], dvr[b],
                              sem_ir[b % 2]).wait()

      @pl.when(t >= 2)
      def _():
        pltpu.make_async_copy(ones_z, cnt_sh.at[dvr[(b + 2) % 4]],
                              sem_s[(b + 2) % 4]).wait()

      pltpu.async_copy(ones_z, cnt_sh.at[dvr[b]], sem_s[b], add=True)

      @pl.when(t < RW - 2)
      def _():
        pltpu.async_copy(dst2d.at[wid + (t + 2) * NW], dvr[(b + 2) % 4],
                         sem_ir[b % 2])

  pltpu.make_async_copy(ones_z, cnt_sh.at[dvr[2]], sem_s[2]).wait()
  pltpu.make_async_copy(ones_z, cnt_sh.at[dvr[3]], sem_s[3]).wait()
  plsc.subcore_barrier()
  _drain(cnt_sh, cnt_out, c, s)


@functools.partial(
    pl.kernel,
    out_type=jax.ShapeDtypeStruct((NC, N, H), jnp.float32),
    mesh=_mesh,
    scratch_types=[
        [pltpu.VMEM((RL,), jnp.int32)] * 4,
        [pltpu.VMEM((RL,), jnp.int32)] * 2,
        pltpu.VMEM((RL, H), jnp.float32),
        pltpu.VMEM((RL, H), jnp.float32),
        pltpu.VMEM((RL, H), jnp.float32),
        pltpu.VMEM_SHARED((N_SH, H), jnp.float32),
        pltpu.SemaphoreType.DMA,
        pltpu.SemaphoreType.DMA,
        pltpu.SemaphoreType.DMA,
        [pltpu.SemaphoreType.DMA] * 2,
    ],
)
def _sc_messages(u_hbm, v_hbm, dst2d, src2d, g_out, dvr, svr, ub, vb, mb,
                 g_sh, sem_u, sem_v, sem_s, sem_ir):
  c = lax.axis_index("c")
  s = lax.axis_index("s")
  wid = s * NC + c

  _fill(mb, RL, 0.0)
  for r5 in range(5):
    pltpu.sync_copy(mb.at[pl.ds(0, RL)],
                    g_sh.at[pl.ds(s * RPT + r5 * RL, RL)])
  plsc.subcore_barrier()

  pltpu.sync_copy(dst2d.at[wid], dvr[0])
  pltpu.sync_copy(src2d.at[wid], svr[0])
  pltpu.async_copy(dst2d.at[wid + NW], dvr[1], sem_ir[1])
  pltpu.async_copy(src2d.at[wid + NW], svr[1], sem_ir[1])

  @pl.loop(0, RW, step=4)
  def _(tt):
    for b in range(4):
      j = tt + b

      @pl.when(j >= 1)
      def _():
        pltpu.make_async_copy(dst2d.at[wid + j * NW], dvr[b],
                              sem_ir[b % 2]).wait()
        pltpu.make_async_copy(src2d.at[wid + j * NW], svr[b % 2],
                              sem_ir[b % 2]).wait()

      cp_u = pltpu.async_copy(u_hbm.at[dvr[b]], ub, sem_u)
      cp_v = pltpu.async_copy(v_hbm.at[svr[b % 2]], vb, sem_v)

      cp_u.wait()
      cp_v.wait()

      @pl.loop(0, RL, unroll=2)
      def _(r):
        for k in range(H // 16):
          sl = pl.ds(k * 16, 16)
          mb[r, sl] = _gelu16(ub[r, sl] + vb[r, sl])

      pltpu.sync_copy(mb, g_sh.at[dvr[b]], add=True)

      @pl.when(j < RW - 2)
      def _():
        c2 = wid + (j + 2) * NW
        pltpu.async_copy(dst2d.at[c2], dvr[(b + 2) % 4], sem_ir[b % 2])
        pltpu.async_copy(src2d.at[c2], svr[b % 2], sem_ir[b % 2])

  plsc.subcore_barrier()
  _drain(g_sh, g_out, c, s)


R = 2000
GRID = N // R

def _full(shape):
  return pl.BlockSpec(shape, lambda i: tuple(0 for _ in shape))


def _tc_pre_body(x_r, pos_r, wi1, bi1, wi2, bi2, w1a, w1b, w1c, b1e,
                 h_r, u_r, v_r):
  f32 = jnp.float32
  h = jnp.dot(x_r[...], wi1[...], preferred_element_type=f32) + bi1[...]
  h = jnp.dot(jax.nn.gelu(h), wi2[...], preferred_element_type=f32) + bi2[...]
  p = jnp.dot(pos_r[...], w1c[...], preferred_element_type=f32)
  h_r[...] = h
  u_r[...] = jnp.dot(h, w1a[...], preferred_element_type=f32) - p + b1e[...]
  v_r[...] = jnp.dot(h, w1b[...], preferred_element_type=f32) + p


_tc_pre = pl.pallas_call(
    _tc_pre_body,
    grid=(GRID,),
    in_specs=[
        pl.BlockSpec((R, H), lambda i: (i, 0)),
        pl.BlockSpec((R, 8), lambda i: (i, 0)),
        _full((H, H)), _full((1, H)), _full((H, H)), _full((1, H)),
        _full((H, H)), _full((H, H)), _full((8, H)), _full((1, H)),
    ],
    out_specs=[
        pl.BlockSpec((R, H), lambda i: (i, 0)),
        pl.BlockSpec((R, H), lambda i: (i, 0)),
        pl.BlockSpec((R, H), lambda i: (i, 0)),
    ],
    out_shape=[jax.ShapeDtypeStruct((N, H), jnp.float32)] * 3,
)


def _node_update(h_r, g2_r, cnt2_r, w2e, b2e, wn1a, wn1b, b1n, wn2, b2n,
                 lng, lnb):
  f32 = jnp.float32
  g = g2_r[0] + g2_r[1]
  cnt = cnt2_r[0, :, 0:1] + cnt2_r[1, :, 0:1]
  aggs = jnp.dot(g, w2e[...], preferred_element_type=f32) + cnt * b2e[...]
  agg = aggs / jnp.maximum(cnt, 1.0)
  hh = h_r[...]
  o = (jnp.dot(hh, wn1a[...], preferred_element_type=f32)
       + jnp.dot(agg, wn1b[...], preferred_element_type=f32) + b1n[...])
  o = jnp.dot(jax.nn.gelu(o), wn2[...], preferred_element_type=f32) + b2n[...]
  hr = hh + o
  mu = jnp.mean(hr, axis=-1, keepdims=True)
  dif = hr - mu
  var = jnp.mean(dif * dif, axis=-1, keepdims=True)
  return dif * jax.lax.rsqrt(var + 1e-5) * lng[...] + lnb[...]


def _tc_mid_body(h_r, g2_r, cnt2_r, pos_r, w2e, b2e, wn1a, wn1b, b1n, wn2,
                 b2n, lng, lnb, w1a, w1b, w1c, b1e, h_out, u_out, v_out):
  f32 = jnp.float32
  hn = _node_update(h_r, g2_r, cnt2_r, w2e, b2e, wn1a, wn1b, b1n, wn2, b2n,
                    lng, lnb)
  p = jnp.dot(pos_r[...], w1c[...], preferred_element_type=f32)
  h_out[...] = hn
  u_out[...] = jnp.dot(hn, w1a[...], preferred_element_type=f32) - p + b1e[...]
  v_out[...] = jnp.dot(hn, w1b[...], preferred_element_type=f32) + p


_tc_mid = pl.pallas_call(
    _tc_mid_body,
    grid=(GRID,),
    in_specs=[
        pl.BlockSpec((R, H), lambda i: (i, 0)),
        pl.BlockSpec((NC, R, H), lambda i: (0, i, 0)),
        pl.BlockSpec((NC, R, 8), lambda i: (0, i, 0)),
        pl.BlockSpec((R, 8), lambda i: (i, 0)),
        _full((H, H)), _full((1, H)), _full((H, H)), _full((H, H)),
        _full((1, H)), _full((H, H)), _full((1, H)), _full((1, H)),
        _full((1, H)),
        _full((H, H)), _full((H, H)), _full((8, H)), _full((1, H)),
    ],
    out_specs=[
        pl.BlockSpec((R, H), lambda i: (i, 0)),
        pl.BlockSpec((R, H), lambda i: (i, 0)),
        pl.BlockSpec((R, H), lambda i: (i, 0)),
    ],
    out_shape=[jax.ShapeDtypeStruct((N, H), jnp.float32)] * 3,
)


def _tc_final_body(h_r, g2_r, cnt2_r, w2e, b2e, wn1a, wn1b, b1n, wn2, b2n,
                   lng, lnb, wh1, bh1, wh2, bh2, pred_out):
  f32 = jnp.float32
  hn = _node_update(h_r, g2_r, cnt2_r, w2e, b2e, wn1a, wn1b, b1n, wn2, b2n,
                    lng, lnb)
  q = jnp.dot(hn, wh1[...], preferred_element_type=f32) + bh1[...]
  q = jax.nn.gelu(q)
  pred_out[...] = jnp.dot(q, wh2[...], preferred_element_type=f32) + bh2[...]


_tc_final = pl.pallas_call(
    _tc_final_body,
    grid=(GRID,),
    in_specs=[
        pl.BlockSpec((R, H), lambda i: (i, 0)),
        pl.BlockSpec((NC, R, H), lambda i: (0, i, 0)),
        pl.BlockSpec((NC, R, 8), lambda i: (0, i, 0)),
        _full((H, H)), _full((1, H)), _full((H, H)), _full((H, H)),
        _full((1, H)), _full((H, H)), _full((1, H)), _full((1, H)),
        _full((1, H)),
        _full((H, H)), _full((1, H)), _full((H, 1)), _full((1, 1)),
    ],
    out_specs=[pl.BlockSpec((R, 1), lambda i: (i, 0))],
    out_shape=[jax.ShapeDtypeStruct((N, 1), jnp.float32)],
)


def _row(v):
  return v.reshape(1, -1)


def kernel(x, edge_index, pos, params):
  dst2d = jnp.concatenate(
      [edge_index[1], jnp.full((EPAD,), N, jnp.int32)]).reshape(EROWS, RL)
  src2d = jnp.concatenate(
      [edge_index[0], jnp.zeros((EPAD,), jnp.int32)]).reshape(EROWS, RL)
  pos8 = jnp.pad(pos, ((0, 0), (0, 6)))

  cnt2 = _sc_degree(dst2d)[:, :, :8]

  blocks = params["blocks"]

  def edge_w(blk):
    w1 = blk["edge"]["W1"]
    w1c8 = jnp.pad(w1[2 * H:], ((0, 6), (0, 0)))
    return w1[:H], w1[H:2 * H], w1c8, _row(blk["edge"]["b1"])

  ip = params["input_proj"]
  w1a, w1b, w1c8, b1e = edge_w(blocks[0])
  h, u, v = _tc_pre(x, pos8, ip["W1"], _row(ip["b1"]), ip["W2"],
                    _row(ip["b2"]), w1a, w1b, w1c8, b1e)

  for s in range(len(blocks)):
    blk = blocks[s]
    g2 = _sc_messages(u, v, dst2d, src2d)
    nw = blk["node"]
    step_w = (blk["edge"]["W2"], _row(blk["edge"]["b2"]),
              nw["W1"][:H], nw["W1"][H:], _row(nw["b1"]), nw["W2"],
              _row(nw["b2"]), _row(blk["ln_g"]), _row(blk["ln_b"]))
    if s + 1 < len(blocks):
      w1a, w1b, w1c8, b1e = edge_w(blocks[s + 1])
      h, u, v = _tc_mid(h, g2, cnt2, pos8, *step_w, w1a, w1b, w1c8, b1e)
    else:
      hd = params["head"]
      (pred,) = _tc_final(h, g2, cnt2, *step_w, hd["W1"], _row(hd["b1"]),
                          hd["W2"], _row(hd["b2"]))
  return pred


# speedup vs baseline: 3.4165x; 3.4165x over previous
import functools

import jax
import jax.numpy as jnp
from jax import lax
from jax.experimental import pallas as pl
from jax.experimental.pallas import tpu as pltpu
from jax.experimental.pallas import tpu_sc as plsc

N = 10000
E = 320000
H = 128
NC = 2
NS = 16
NW = NC * NS
RL = 128
CH = 64
RW = 80
EROWS = NW * RW // 2 * 2
EROWS = 2560
EPAD = EROWS * RL - E
N_SH = N + 8
RPT = 624
RPT_LAST = N - 15 * RPT
ZR = 64

_K1 = -2.0 * 0.7978845608028654
_K2 = _K1 * 0.044715

_mesh = plsc.VectorSubcoreMesh(
    core_axis_name="c", subcore_axis_name="s", num_cores=NC, num_subcores=NS)


def _gelu16(t):
  d = jnp.exp(_K1 * t + _K2 * (t * t * t))
  return t / (1.0 + d)


def _drain(sh, out, c, s):
  @pl.when(s < NS - 1)
  def _():
    pltpu.sync_copy(sh.at[pl.ds(s * RPT, RPT)],
                    out.at[c, pl.ds(s * RPT, RPT)])

  @pl.when(s == NS - 1)
  def _():
    pltpu.sync_copy(sh.at[pl.ds(15 * RPT, RPT_LAST)],
                    out.at[c, pl.ds(15 * RPT, RPT_LAST)])


def _fill(buf, rows, value):
  @pl.loop(0, rows)
  def _(i):
    for k in range(H // 16):
      buf[i, pl.ds(k * 16, 16)] = jnp.full((16,), value, jnp.float32)


def _split_row(row, a64, b64):
  for k in range(4):
    a64[pl.ds(k * 16, 16)] = row[pl.ds(k * 16, 16)]
  for k in range(4):
    b64[pl.ds(k * 16, 16)] = row[pl.ds(64 + k * 16, 16)]


@functools.partial(
    pl.kernel,
    out_type=jax.ShapeDtypeStruct((NC, N, H), jnp.float32),
    mesh=_mesh,
    scratch_types=[
        [pltpu.VMEM((RL,), jnp.int32)] * 4,
        pltpu.VMEM((RL, H), jnp.float32),
        pltpu.VMEM_SHARED((N_SH, H), jnp.float32),
        [pltpu.SemaphoreType.DMA] * 4,
        [pltpu.SemaphoreType.DMA] * 2,
    ],
)
def _sc_degree(dst2d, cnt_out, dvr, ones_z, cnt_sh, sem_s, sem_ir):
  c = lax.axis_index("c")
  s = lax.axis_index("s")
  wid = s * NC + c

  _fill(ones_z, ZR, 0.0)
  for r5 in range(10):
    pltpu.sync_copy(ones_z.at[pl.ds(0, ZR)],
                    cnt_sh.at[pl.ds(s * RPT + r5 * ZR, ZR)])
  _fill(ones_z, RL, 1.0)
  plsc.subcore_barrier()

  pltpu.sync_copy(dst2d.at[wid], dvr[0])
  pltpu.async_copy(dst2d.at[wid + NW], dvr[1], sem_ir[1])

  @pl.loop(0, RW, step=4)
  def _(tt):
    for b in range(4):
      t = tt + b

      @pl.when(t >= 1)
      def _():
        pltpu.make_async_copy(dst2d.at[wid + t * NW], dvr[b],
                              sem_ir[b % 2]).wait()

      @pl.when(t >= 2)
      def _():
        pltpu.make_async_copy(ones_z, cnt_sh.at[dvr[(b + 2) % 4]],
                              sem_s[(b + 2) % 4]).wait()

      pltpu.async_copy(ones_z, cnt_sh.at[dvr[b]], sem_s[b], add=True)

      @pl.when(t < RW - 2)
      def _():
        pltpu.async_copy(dst2d.at[wid + (t + 2) * NW], dvr[(b + 2) % 4],
                         sem_ir[b % 2])

  pltpu.make_async_copy(ones_z, cnt_sh.at[dvr[2]], sem_s[2]).wait()
  pltpu.make_async_copy(ones_z, cnt_sh.at[dvr[3]], sem_s[3]).wait()
  plsc.subcore_barrier()
  _drain(cnt_sh, cnt_out, c, s)


@functools.partial(
    pl.kernel,
    out_type=jax.ShapeDtypeStruct((NC, N, H), jnp.float32),
    mesh=_mesh,
    scratch_types=[
        [pltpu.VMEM((RL,), jnp.int32)] * 4,
        [pltpu.VMEM((RL,), jnp.int32)] * 2,
        pltpu.VMEM((RL, H), jnp.float32),
        pltpu.VMEM((RL, H), jnp.float32),
        pltpu.VMEM((RL, H), jnp.float32),
        pltpu.VMEM_SHARED((N_SH, H), jnp.float32),
        pltpu.SemaphoreType.DMA,
        pltpu.SemaphoreType.DMA,
        pltpu.SemaphoreType.DMA,
        [pltpu.SemaphoreType.DMA] * 2,
    ],
)
def _sc_messages(u_hbm, v_hbm, dst2d, src2d, g_out, dvr, svr, ub, vb, mb,
                 g_sh, sem_u, sem_v, sem_s, sem_ir):
  c = lax.axis_index("c")
  s = lax.axis_index("s")
  wid = s * NC + c

  _fill(mb, RL, 0.0)
  for r5 in range(5):
    pltpu.sync_copy(mb.at[pl.ds(0, RL)],
                    g_sh.at[pl.ds(s * RPT + r5 * RL, RL)])
  plsc.subcore_barrier()

  pltpu.sync_copy(dst2d.at[wid], dvr[0])
  pltpu.sync_copy(src2d.at[wid], svr[0])
  pltpu.async_copy(dst2d.at[wid + NW], dvr[1], sem_ir[1])
  pltpu.async_copy(src2d.at[wid + NW], svr[1], sem_ir[1])

  @pl.loop(0, RW, step=4)
  def _(tt):
    for b in range(4):
      j = tt + b

      @pl.when(j >= 1)
      def _():
        pltpu.make_async_copy(dst2d.at[wid + j * NW], dvr[b],
                              sem_ir[b % 2]).wait()
        pltpu.make_async_copy(src2d.at[wid + j * NW], svr[b % 2],
                              sem_ir[b % 2]).wait()

      cp_u = pltpu.async_copy(u_hbm.at[dvr[b]], ub, sem_u)
      cp_v = pltpu.async_copy(v_hbm.at[svr[b % 2]], vb, sem_v)

      cp_u.wait()
      cp_v.wait()

      @pl.loop(0, RL)
      def _(r):
        for k in range(H // 16):
          sl = pl.ds(k * 16, 16)
          mb[r, sl] = _gelu16(ub[r, sl] + vb[r, sl])

      pltpu.sync_copy(mb, g_sh.at[dvr[b]], add=True)

      @pl.when(j < RW - 2)
      def _():
        c2 = wid + (j + 2) * NW
        pltpu.async_copy(dst2d.at[c2], dvr[(b + 2) % 4], sem_ir[b % 2])
        pltpu.async_copy(src2d.at[c2], svr[b % 2], sem_ir[b % 2])

  plsc.subcore_barrier()
  _drain(g_sh, g_out, c, s)


R = 2000
GRID = N // R

def _full(shape):
  return pl.BlockSpec(shape, lambda i: tuple(0 for _ in shape))


def _tc_pre_body(x_r, pos_r, wi1, bi1, wi2, bi2, w1a, w1b, w1c, b1e,
                 h_r, u_r, v_r):
  f32 = jnp.float32
  h = jnp.dot(x_r[...], wi1[...], preferred_element_type=f32) + bi1[...]
  h = jnp.dot(jax.nn.gelu(h), wi2[...], preferred_element_type=f32) + bi2[...]
  p = jnp.dot(pos_r[...], w1c[...], preferred_element_type=f32)
  h_r[...] = h
  u_r[...] = jnp.dot(h, w1a[...], preferred_element_type=f32) - p + b1e[...]
  v_r[...] = jnp.dot(h, w1b[...], preferred_element_type=f32) + p


_tc_pre = pl.pallas_call(
    _tc_pre_body,
    grid=(GRID,),
    in_specs=[
        pl.BlockSpec((R, H), lambda i: (i, 0)),
        pl.BlockSpec((R, 8), lambda i: (i, 0)),
        _full((H, H)), _full((1, H)), _full((H, H)), _full((1, H)),
        _full((H, H)), _full((H, H)), _full((8, H)), _full((1, H)),
    ],
    out_specs=[
        pl.BlockSpec((R, H), lambda i: (i, 0)),
        pl.BlockSpec((R, H), lambda i: (i, 0)),
        pl.BlockSpec((R, H), lambda i: (i, 0)),
    ],
    out_shape=[jax.ShapeDtypeStruct((N, H), jnp.float32)] * 3,
)


def _node_update(h_r, g2_r, cnt2_r, w2e, b2e, wn1a, wn1b, b1n, wn2, b2n,
                 lng, lnb):
  f32 = jnp.float32
  g = g2_r[0] + g2_r[1]
  cnt = cnt2_r[0, :, 0:1] + cnt2_r[1, :, 0:1]
  aggs = jnp.dot(g, w2e[...], preferred_element_type=f32) + cnt * b2e[...]
  agg = aggs / jnp.maximum(cnt, 1.0)
  hh = h_r[...]
  o = (jnp.dot(hh, wn1a[...], preferred_element_type=f32)
       + jnp.dot(agg, wn1b[...], preferred_element_type=f32) + b1n[...])
  o = jnp.dot(jax.nn.gelu(o), wn2[...], preferred_element_type=f32) + b2n[...]
  hr = hh + o
  mu = jnp.mean(hr, axis=-1, keepdims=True)
  dif = hr - mu
  var = jnp.mean(dif * dif, axis=-1, keepdims=True)
  return dif * jax.lax.rsqrt(var + 1e-5) * lng[...] + lnb[...]


def _tc_mid_body(h_r, g2_r, cnt2_r, pos_r, w2e, b2e, wn1a, wn1b, b1n, wn2,
                 b2n, lng, lnb, w1a, w1b, w1c, b1e, h_out, u_out, v_out):
  f32 = jnp.float32
  hn = _node_update(h_r, g2_r, cnt2_r, w2e, b2e, wn1a, wn1b, b1n, wn2, b2n,
                    lng, lnb)
  p = jnp.dot(pos_r[...], w1c[...], preferred_element_type=f32)
  h_out[...] = hn
  u_out[...] = jnp.dot(hn, w1a[...], preferred_element_type=f32) - p + b1e[...]
  v_out[...] = jnp.dot(hn, w1b[...], preferred_element_type=f32) + p


_tc_mid = pl.pallas_call(
    _tc_mid_body,
    grid=(GRID,),
    in_specs=[
        pl.BlockSpec((R, H), lambda i: (i, 0)),
        pl.BlockSpec((NC, R, H), lambda i: (0, i, 0)),
        pl.BlockSpec((NC, R, 8), lambda i: (0, i, 0)),
        pl.BlockSpec((R, 8), lambda i: (i, 0)),
        _full((H, H)), _full((1, H)), _full((H, H)), _full((H, H)),
        _full((1, H)), _full((H, H)), _full((1, H)), _full((1, H)),
        _full((1, H)),
        _full((H, H)), _full((H, H)), _full((8, H)), _full((1, H)),
    ],
    out_specs=[
        pl.BlockSpec((R, H), lambda i: (i, 0)),
        pl.BlockSpec((R, H), lambda i: (i, 0)),
        pl.BlockSpec((R, H), lambda i: (i, 0)),
    ],
    out_shape=[jax.ShapeDtypeStruct((N, H), jnp.float32)] * 3,
)


def _tc_final_body(h_r, g2_r, cnt2_r, w2e, b2e, wn1a, wn1b, b1n, wn2, b2n,
                   lng, lnb, wh1, bh1, wh2, bh2, pred_out):
  f32 = jnp.float32
  hn = _node_update(h_r, g2_r, cnt2_r, w2e, b2e, wn1a, wn1b, b1n, wn2, b2n,
                    lng, lnb)
  q = jnp.dot(hn, wh1[...], preferred_element_type=f32) + bh1[...]
  q = jax.nn.gelu(q)
  pred_out[...] = jnp.dot(q, wh2[...], preferred_element_type=f32) + bh2[...]


_tc_final = pl.pallas_call(
    _tc_final_body,
    grid=(GRID,),
    in_specs=[
        pl.BlockSpec((R, H), lambda i: (i, 0)),
        pl.BlockSpec((NC, R, H), lambda i: (0, i, 0)),
        pl.BlockSpec((NC, R, 8), lambda i: (0, i, 0)),
        _full((H, H)), _full((1, H)), _full((H, H)), _full((H, H)),
        _full((1, H)), _full((H, H)), _full((1, H)), _full((1, H)),
        _full((1, H)),
        _full((H, H)), _full((1, H)), _full((H, 1)), _full((1, 1)),
    ],
    out_specs=[pl.BlockSpec((R, 1), lambda i: (i, 0))],
    out_shape=[jax.ShapeDtypeStruct((N, 1), jnp.float32)],
)


def _row(v):
  return v.reshape(1, -1)


def kernel(x, edge_index, pos, params):
  dst2d = jnp.concatenate(
      [edge_index[1], jnp.full((EPAD,), N, jnp.int32)]).reshape(EROWS, RL)
  src2d = jnp.concatenate(
      [edge_index[0], jnp.zeros((EPAD,), jnp.int32)]).reshape(EROWS, RL)
  pos8 = jnp.pad(pos, ((0, 0), (0, 6)))

  cnt2 = _sc_degree(dst2d)[:, :, :8]

  blocks = params["blocks"]

  def edge_w(blk):
    w1 = blk["edge"]["W1"]
    w1c8 = jnp.pad(w1[2 * H:], ((0, 6), (0, 0)))
    return w1[:H], w1[H:2 * H], w1c8, _row(blk["edge"]["b1"])

  ip = params["input_proj"]
  w1a, w1b, w1c8, b1e = edge_w(blocks[0])
  h, u, v = _tc_pre(x, pos8, ip["W1"], _row(ip["b1"]), ip["W2"],
                    _row(ip["b2"]), w1a, w1b, w1c8, b1e)

  for s in range(len(blocks)):
    blk = blocks[s]
    g2 = _sc_messages(u, v, dst2d, src2d)
    nw = blk["node"]
    step_w = (blk["edge"]["W2"], _row(blk["edge"]["b2"]),
              nw["W1"][:H], nw["W1"][H:], _row(nw["b1"]), nw["W2"],
              _row(nw["b2"]), _row(blk["ln_g"]), _row(blk["ln_b"]))
    if s + 1 < len(blocks):
      w1a, w1b, w1c8, b1e = edge_w(blocks[s + 1])
      h, u, v = _tc_mid(h, g2, cnt2, pos8, *step_w, w1a, w1b, w1c8, b1e)
    else:
      hd = params["head"]
      (pred,) = _tc_final(h, g2, cnt2, *step_w, hd["W1"], _row(hd["b1"]),
                          hd["W2"], _row(hd["b2"]))
  return pred


# speedup vs baseline: 3.6321x; 1.0631x over previous
import functools

import jax
import jax.numpy as jnp
from jax import lax
from jax.experimental import pallas as pl
from jax.experimental.pallas import tpu as pltpu
from jax.experimental.pallas import tpu_sc as plsc

N = 10000
E = 320000
H = 128
NC = 2
NS = 16
NW = NC * NS
RL = 128
CH = 64
RW = 80
EROWS = NW * RW // 2 * 2
EROWS = 2560
EPAD = EROWS * RL - E
N_SH = N + 8
RPT = 624
RPT_LAST = N - 15 * RPT
ZR = 64

_K1 = -2.0 * 0.7978845608028654
_K2 = _K1 * 0.044715

_mesh = plsc.VectorSubcoreMesh(
    core_axis_name="c", subcore_axis_name="s", num_cores=NC, num_subcores=NS)


def _gelu16(t):
  d = jnp.exp(_K1 * t + _K2 * (t * t * t))
  return t / (1.0 + d)


def _drain(sh, out, c, s):
  @pl.when(s < NS - 1)
  def _():
    pltpu.sync_copy(sh.at[pl.ds(s * RPT, RPT)],
                    out.at[c, pl.ds(s * RPT, RPT)])

  @pl.when(s == NS - 1)
  def _():
    pltpu.sync_copy(sh.at[pl.ds(15 * RPT, RPT_LAST)],
                    out.at[c, pl.ds(15 * RPT, RPT_LAST)])


def _fill(buf, rows, value):
  @pl.loop(0, rows)
  def _(i):
    for k in range(H // 16):
      buf[i, pl.ds(k * 16, 16)] = jnp.full((16,), value, jnp.float32)


def _split_row(row, a64, b64):
  for k in range(4):
    a64[pl.ds(k * 16, 16)] = row[pl.ds(k * 16, 16)]
  for k in range(4):
    b64[pl.ds(k * 16, 16)] = row[pl.ds(64 + k * 16, 16)]


@functools.partial(
    pl.kernel,
    out_type=jax.ShapeDtypeStruct((NC, N, H), jnp.float32),
    mesh=_mesh,
    scratch_types=[
        [pltpu.VMEM((RL,), jnp.int32)] * 4,
        pltpu.VMEM((RL, H), jnp.float32),
        pltpu.VMEM_SHARED((N_SH, H), jnp.float32),
        [pltpu.SemaphoreType.DMA] * 4,
        [pltpu.SemaphoreType.DMA] * 2,
    ],
)
def _sc_degree(dst2d, cnt_out, dvr, ones_z, cnt_sh, sem_s, sem_ir):
  c = lax.axis_index("c")
  s = lax.axis_index("s")
  wid = s * NC + c

  _fill(ones_z, ZR, 0.0)
  for r5 in range(10):
    pltpu.sync_copy(ones_z.at[pl.ds(0, ZR)],
                    cnt_sh.at[pl.ds(s * RPT + r5 * ZR, ZR)])
  _fill(ones_z, RL, 1.0)
  plsc.subcore_barrier()

  pltpu.sync_copy(dst2d.at[wid], dvr[0])
  pltpu.async_copy(dst2d.at[wid + NW], dvr[1], sem_ir[1])

  @pl.loop(0, RW, step=4)
  def _(tt):
    for b in range(4):
      t = tt + b

      @pl.when(t >= 1)
      def _():
        pltpu.make_async_copy(dst2d.at[wid + t * NW], dvr[b],
                              sem_ir[b % 2]).wait()

      @pl.when(t >= 2)
      def _():
        pltpu.make_async_copy(ones_z, cnt_sh.at[dvr[(b + 2) % 4]],
                              sem_s[(b + 2) % 4]).wait()

      pltpu.async_copy(ones_z, cnt_sh.at[dvr[b]], sem_s[b], add=True)

      @pl.when(t < RW - 2)
      def _():
        pltpu.async_copy(dst2d.at[wid + (t + 2) * NW], dvr[(b + 2) % 4],
                         sem_ir[b % 2])

  pltpu.make_async_copy(ones_z, cnt_sh.at[dvr[2]], sem_s[2]).wait()
  pltpu.make_async_copy(ones_z, cnt_sh.at[dvr[3]], sem_s[3]).wait()
  plsc.subcore_barrier()
  _drain(cnt_sh, cnt_out, c, s)


@functools.partial(
    pl.kernel,
    out_type=jax.ShapeDtypeStruct((NC, N, H), jnp.float32),
    mesh=_mesh,
    scratch_types=[
        [pltpu.VMEM((RL,), jnp.int32)] * 4,
        [pltpu.VMEM((RL,), jnp.int32)] * 2,
        pltpu.VMEM((RL, H), jnp.float32),
        pltpu.VMEM((RL, H), jnp.float32),
        pltpu.VMEM((RL, H), jnp.float32),
        pltpu.VMEM_SHARED((N_SH, H), jnp.float32),
        pltpu.SemaphoreType.DMA,
        pltpu.SemaphoreType.DMA,
        pltpu.SemaphoreType.DMA,
        [pltpu.SemaphoreType.DMA] * 2,
    ],
)
def _sc_messages(u_hbm, v_hbm, dst2d, src2d, g_out, dvr, svr, ub, vb, mb,
                 g_sh, sem_u, sem_v, sem_s, sem_ir):
  c = lax.axis_index("c")
  s = lax.axis_index("s")
  wid = s * NC + c

  _fill(mb, RL, 0.0)
  for r5 in range(5):
    pltpu.sync_copy(mb.at[pl.ds(0, RL)],
                    g_sh.at[pl.ds(s * RPT + r5 * RL, RL)])
  plsc.subcore_barrier()

  pltpu.sync_copy(dst2d.at[wid], dvr[0])
  pltpu.sync_copy(src2d.at[wid], svr[0])
  pltpu.async_copy(dst2d.at[wid + NW], dvr[1], sem_ir[1])
  pltpu.async_copy(src2d.at[wid + NW], svr[1], sem_ir[1])

  @pl.loop(0, RW, step=4)
  def _(tt):
    for b in range(4):
      j = tt + b

      @pl.when(j >= 1)
      def _():
        pltpu.make_async_copy(dst2d.at[wid + j * NW], dvr[b],
                              sem_ir[b % 2]).wait()
        pltpu.make_async_copy(src2d.at[wid + j * NW], svr[b % 2],
                              sem_ir[b % 2]).wait()

      cp_u = pltpu.async_copy(u_hbm.at[dvr[b]], ub, sem_u)
      cp_v = pltpu.async_copy(v_hbm.at[svr[b % 2]], vb, sem_v)

      @pl.when(j >= 1)
      def _():
        pltpu.make_async_copy(mb, g_sh.at[dvr[(b + 3) % 4]], sem_s).wait()

      cp_u.wait()
      cp_v.wait()

      @pl.loop(0, RL)
      def _(r):
        for k in range(H // 16):
          sl = pl.ds(k * 16, 16)
          mb[r, sl] = _gelu16(ub[r, sl] + vb[r, sl])

      pltpu.async_copy(mb, g_sh.at[dvr[b]], sem_s, add=True)

      @pl.when(j < RW - 2)
      def _():
        c2 = wid + (j + 2) * NW
        pltpu.async_copy(dst2d.at[c2], dvr[(b + 2) % 4], sem_ir[b % 2])
        pltpu.async_copy(src2d.at[c2], svr[b % 2], sem_ir[b % 2])

  pltpu.make_async_copy(mb, g_sh.at[dvr[(RW - 1) % 4]], sem_s).wait()
  plsc.subcore_barrier()
  _drain(g_sh, g_out, c, s)


R = 2000
GRID = N // R

def _full(shape):
  return pl.BlockSpec(shape, lambda i: tuple(0 for _ in shape))


def _tc_pre_body(x_r, pos_r, wi1, bi1, wi2, bi2, w1a, w1b, w1c, b1e,
                 h_r, u_r, v_r):
  f32 = jnp.float32
  h = jnp.dot(x_r[...], wi1[...], preferred_element_type=f32) + bi1[...]
  h = jnp.dot(jax.nn.gelu(h), wi2[...], preferred_element_type=f32) + bi2[...]
  p = jnp.dot(pos_r[...], w1c[...], preferred_element_type=f32)
  h_r[...] = h
  u_r[...] = jnp.dot(h, w1a[...], preferred_element_type=f32) - p + b1e[...]
  v_r[...] = jnp.dot(h, w1b[...], preferred_element_type=f32) + p


_tc_pre = pl.pallas_call(
    _tc_pre_body,
    grid=(GRID,),
    in_specs=[
        pl.BlockSpec((R, H), lambda i: (i, 0)),
        pl.BlockSpec((R, 8), lambda i: (i, 0)),
        _full((H, H)), _full((1, H)), _full((H, H)), _full((1, H)),
        _full((H, H)), _full((H, H)), _full((8, H)), _full((1, H)),
    ],
    out_specs=[
        pl.BlockSpec((R, H), lambda i: (i, 0)),
        pl.BlockSpec((R, H), lambda i: (i, 0)),
        pl.BlockSpec((R, H), lambda i: (i, 0)),
    ],
    out_shape=[jax.ShapeDtypeStruct((N, H), jnp.float32)] * 3,
)


def _node_update(h_r, g2_r, cnt2_r, w2e, b2e, wn1a, wn1b, b1n, wn2, b2n,
                 lng, lnb):
  f32 = jnp.float32
  g = g2_r[0] + g2_r[1]
  cnt = cnt2_r[0, :, 0:1] + cnt2_r[1, :, 0:1]
  aggs = jnp.dot(g, w2e[...], preferred_element_type=f32) + cnt * b2e[...]
  agg = aggs / jnp.maximum(cnt, 1.0)
  hh = h_r[...]
  o = (jnp.dot(hh, wn1a[...], preferred_element_type=f32)
       + jnp.dot(agg, wn1b[...], preferred_element_type=f32) + b1n[...])
  o = jnp.dot(jax.nn.gelu(o), wn2[...], preferred_element_type=f32) + b2n[...]
  hr = hh + o
  mu = jnp.mean(hr, axis=-1, keepdims=True)
  dif = hr - mu
  var = jnp.mean(dif * dif, axis=-1, keepdims=True)
  return dif * jax.lax.rsqrt(var + 1e-5) * lng[...] + lnb[...]


def _tc_mid_body(h_r, g2_r, cnt2_r, pos_r, w2e, b2e, wn1a, wn1b, b1n, wn2,
                 b2n, lng, lnb, w1a, w1b, w1c, b1e, h_out, u_out, v_out):
  f32 = jnp.float32
  hn = _node_update(h_r, g2_r, cnt2_r, w2e, b2e, wn1a, wn1b, b1n, wn2, b2n,
                    lng, lnb)
  p = jnp.dot(pos_r[...], w1c[...], preferred_element_type=f32)
  h_out[...] = hn
  u_out[...] = jnp.dot(hn, w1a[...], preferred_element_type=f32) - p + b1e[...]
  v_out[...] = jnp.dot(hn, w1b[...], preferred_element_type=f32) + p


_tc_mid = pl.pallas_call(
    _tc_mid_body,
    grid=(GRID,),
    in_specs=[
        pl.BlockSpec((R, H), lambda i: (i, 0)),
        pl.BlockSpec((NC, R, H), lambda i: (0, i, 0)),
        pl.BlockSpec((NC, R, 8), lambda i: (0, i, 0)),
        pl.BlockSpec((R, 8), lambda i: (i, 0)),
        _full((H, H)), _full((1, H)), _full((H, H)), _full((H, H)),
        _full((1, H)), _full((H, H)), _full((1, H)), _full((1, H)),
        _full((1, H)),
        _full((H, H)), _full((H, H)), _full((8, H)), _full((1, H)),
    ],
    out_specs=[
        pl.BlockSpec((R, H), lambda i: (i, 0)),
        pl.BlockSpec((R, H), lambda i: (i, 0)),
        pl.BlockSpec((R, H), lambda i: (i, 0)),
    ],
    out_shape=[jax.ShapeDtypeStruct((N, H), jnp.float32)] * 3,
)


def _tc_final_body(h_r, g2_r, cnt2_r, w2e, b2e, wn1a, wn1b, b1n, wn2, b2n,
                   lng, lnb, wh1, bh1, wh2, bh2, pred_out):
  f32 = jnp.float32
  hn = _node_update(h_r, g2_r, cnt2_r, w2e, b2e, wn1a, wn1b, b1n, wn2, b2n,
                    lng, lnb)
  q = jnp.dot(hn, wh1[...], preferred_element_type=f32) + bh1[...]
  q = jax.nn.gelu(q)
  pred_out[...] = jnp.dot(q, wh2[...], preferred_element_type=f32) + bh2[...]


_tc_final = pl.pallas_call(
    _tc_final_body,
    grid=(GRID,),
    in_specs=[
        pl.BlockSpec((R, H), lambda i: (i, 0)),
        pl.BlockSpec((NC, R, H), lambda i: (0, i, 0)),
        pl.BlockSpec((NC, R, 8), lambda i: (0, i, 0)),
        _full((H, H)), _full((1, H)), _full((H, H)), _full((H, H)),
        _full((1, H)), _full((H, H)), _full((1, H)), _full((1, H)),
        _full((1, H)),
        _full((H, H)), _full((1, H)), _full((H, 1)), _full((1, 1)),
    ],
    out_specs=[pl.BlockSpec((R, 1), lambda i: (i, 0))],
    out_shape=[jax.ShapeDtypeStruct((N, 1), jnp.float32)],
)


def _row(v):
  return v.reshape(1, -1)


def kernel(x, edge_index, pos, params):
  dst2d = jnp.concatenate(
      [edge_index[1], jnp.full((EPAD,), N, jnp.int32)]).reshape(EROWS, RL)
  src2d = jnp.concatenate(
      [edge_index[0], jnp.zeros((EPAD,), jnp.int32)]).reshape(EROWS, RL)
  pos8 = jnp.pad(pos, ((0, 0), (0, 6)))

  cnt2 = _sc_degree(dst2d)[:, :, :8]

  blocks = params["blocks"]

  def edge_w(blk):
    w1 = blk["edge"]["W1"]
    w1c8 = jnp.pad(w1[2 * H:], ((0, 6), (0, 0)))
    return w1[:H], w1[H:2 * H], w1c8, _row(blk["edge"]["b1"])

  ip = params["input_proj"]
  w1a, w1b, w1c8, b1e = edge_w(blocks[0])
  h, u, v = _tc_pre(x, pos8, ip["W1"], _row(ip["b1"]), ip["W2"],
                    _row(ip["b2"]), w1a, w1b, w1c8, b1e)

  for s in range(len(blocks)):
    blk = blocks[s]
    g2 = _sc_messages(u, v, dst2d, src2d)
    nw = blk["node"]
    step_w = (blk["edge"]["W2"], _row(blk["edge"]["b2"]),
              nw["W1"][:H], nw["W1"][H:], _row(nw["b1"]), nw["W2"],
              _row(nw["b2"]), _row(blk["ln_g"]), _row(blk["ln_b"]))
    if s + 1 < len(blocks):
      w1a, w1b, w1c8, b1e = edge_w(blocks[s + 1])
      h, u, v = _tc_mid(h, g2, cnt2, pos8, *step_w, w1a, w1b, w1c8, b1e)
    else:
      hd = params["head"]
      (pred,) = _tc_final(h, g2, cnt2, *step_w, hd["W1"], _row(hd["b1"]),
                          hd["W2"], _row(hd["b2"]))
  return pred
